# initial kernel scaffold (unmeasured)
import jax
import jax.numpy as jnp
from jax import lax
from jax.experimental import pallas as pl
from jax.experimental.pallas import tpu as pltpu

P = 16
M, K, N = 4096, 4096, 2048
MP = M // P
KP = K // P


def kernel(x, w_mat, scale_x, scale_w):
    def body(x_ref, w_ref, sx_ref, sw_ref, out_ref,
             xs_ref, buf_ref, send_sems, recv_sems):
        me = lax.axis_index("i")

        xs_ref[...] = x_ref[...].astype(jnp.float8_e4m3fn)

        rdmas = []
        for off in range(1, P):
            tgt = lax.rem(me + off, P)
            rdma = pltpu.make_async_remote_copy(
                src_ref=xs_ref.at[pl.ds(tgt * MP, MP), :],
                dst_ref=buf_ref.at[off],
                send_sem=send_sems.at[off],
                recv_sem=recv_sems.at[off],
                device_id=(tgt,),
                device_id_type=pl.DeviceIdType.MESH,
            )
            rdma.start()
            rdmas.append(rdma)

        def mm(a_fp8, k_blk):
            wb = w_ref[pl.ds(k_blk * KP, KP), :].astype(jnp.float8_e4m3fn)
            return lax.dot_general(
                a_fp8, wb, (((1,), (0,)), ((), ())),
                preferred_element_type=jnp.float32)

        acc = mm(xs_ref[pl.ds(me * MP, MP), :], me)

        for off in range(1, P):
            rdmas[off - 1].wait_recv()
            src = lax.rem(me - off + P, P)
            acc = acc + mm(buf_ref[off], src)

        for off in range(1, P):
            rdmas[off - 1].wait_send()

        y = acc * (sx_ref[0] * sw_ref[0])
        out_ref[...] = y * jax.nn.sigmoid(jnp.clip(y, -60.0, 60.0))

    return pl.pallas_call(
        body,
        out_shape=jax.ShapeDtypeStruct((MP, N), jnp.float32),
        in_specs=[
            pl.BlockSpec(memory_space=pltpu.VMEM),
            pl.BlockSpec(memory_space=pltpu.VMEM),
            pl.BlockSpec(memory_space=pltpu.SMEM),
            pl.BlockSpec(memory_space=pltpu.SMEM),
        ],
        out_specs=pl.BlockSpec(memory_space=pltpu.VMEM),
        scratch_shapes=[
            pltpu.VMEM((M, KP), jnp.float8_e4m3fn),
            pltpu.VMEM((P, MP, KP), jnp.float8_e4m3fn),
            pltpu.SemaphoreType.DMA((P,)),
            pltpu.SemaphoreType.DMA((P,)),
        ],
        compiler_params=pltpu.CompilerParams(collective_id=0),
    )(x, w_mat, scale_x, scale_w)


# baseline (device time: 32924 ns/iter reference)
import jax
import jax.numpy as jnp
from jax import lax
from jax.experimental import pallas as pl
from jax.experimental.pallas import tpu as pltpu

P = 16
M, K, N = 4096, 4096, 2048
MP = M // P
KP = K // P


def kernel(x, w_mat, scale_x, scale_w):
    def body(x_ref, w_hbm, sx_ref, sw_ref, out_ref,
             xs_ref, buf_ref, wbuf_ref, send_sems, recv_sems, w_sems):
        me = lax.axis_index("i")

        xs_ref[...] = x_ref[...].astype(jnp.float8_e4m3fn)

        wcopies = []

        def start_wcopy(off):
            k0 = lax.rem(me - off + P, P)
            cp = pltpu.make_async_copy(
                w_hbm.at[pl.ds(k0 * KP, KP), :],
                wbuf_ref.at[off % 2],
                w_sems.at[off % 2],
            )
            cp.start()
            wcopies.append(cp)

        start_wcopy(0)

        rdmas = []
        for off in range(1, P):
            tgt = lax.rem(me + off, P)
            rdma = pltpu.make_async_remote_copy(
                src_ref=xs_ref.at[pl.ds(tgt * MP, MP), :],
                dst_ref=buf_ref.at[off],
                send_sem=send_sems.at[off],
                recv_sem=recv_sems.at[off],
                device_id=(tgt,),
                device_id_type=pl.DeviceIdType.MESH,
            )
            rdma.start()
            rdmas.append(rdma)

        acc = None
        for off in range(P):
            if off + 1 < P:
                start_wcopy(off + 1)
            wcopies[off].wait()
            if off == 0:
                a = xs_ref[pl.ds(me * MP, MP), :]
            else:
                rdmas[off - 1].wait_recv()
                a = buf_ref[off]
            wb = wbuf_ref[off % 2].astype(jnp.float8_e4m3fn)
            part = lax.dot_general(
                a, wb, (((1,), (0,)), ((), ())),
                preferred_element_type=jnp.float32)
            acc = part if acc is None else acc + part

        for off in range(1, P):
            rdmas[off - 1].wait_send()

        y = acc * (sx_ref[0] * sw_ref[0])
        out_ref[...] = y * jax.nn.sigmoid(jnp.clip(y, -60.0, 60.0))

    return pl.pallas_call(
        body,
        out_shape=jax.ShapeDtypeStruct((MP, N), jnp.float32),
        in_specs=[
            pl.BlockSpec(memory_space=pltpu.VMEM),
            pl.BlockSpec(memory_space=pl.ANY),
            pl.BlockSpec(memory_space=pltpu.SMEM),
            pl.BlockSpec(memory_space=pltpu.SMEM),
        ],
        out_specs=pl.BlockSpec(memory_space=pltpu.VMEM),
        scratch_shapes=[
            pltpu.VMEM((M, KP), jnp.float8_e4m3fn),
            pltpu.VMEM((P, MP, KP), jnp.float8_e4m3fn),
            pltpu.VMEM((2, KP, N), jnp.float32),
            pltpu.SemaphoreType.DMA((P,)),
            pltpu.SemaphoreType.DMA((P,)),
            pltpu.SemaphoreType.DMA((2,)),
        ],
    )(x, w_mat, scale_x, scale_w)


# device time: 19554 ns/iter; 1.6837x vs baseline; 1.6837x over previous
import os

import jax
import jax.numpy as jnp
from jax import lax
from jax.experimental import pallas as pl
from jax.experimental.pallas import tpu as pltpu

_VARIANT = os.environ.get("KVARIANT", "full")

P = 16
M, K, N = 4096, 4096, 2048
MP = M // P
KP = K // P


def kernel(x, w_mat, scale_x, scale_w):
    def body(x_ref, w_hbm, sx_ref, sw_ref, out_ref,
             xs_ref, buf_ref, wbuf_ref, send_sems, recv_sems, w_sems):
        me = lax.axis_index("i")

        xs_ref[...] = x_ref[...].astype(jnp.float8_e4m3fn)

        wcopies = []

        def start_wcopy(off):
            k0 = lax.rem(me - off + P, P)
            cp = pltpu.make_async_copy(
                w_hbm.at[pl.ds(k0 * KP, KP), :],
                wbuf_ref.at[off % 2],
                w_sems.at[off % 2],
            )
            cp.start()
            wcopies.append(cp)

        start_wcopy(0)

        rdmas = []
        for off in range(1, P) if _VARIANT != "local" else []:
            tgt = lax.rem(me + off, P)
            rdma = pltpu.make_async_remote_copy(
                src_ref=xs_ref.at[pl.ds(tgt * MP, MP), :],
                dst_ref=buf_ref.at[off],
                send_sem=send_sems.at[off],
                recv_sem=recv_sems.at[off],
                device_id=(tgt,),
                device_id_type=pl.DeviceIdType.MESH,
            )
            rdma.start()
            rdmas.append(rdma)

        acc = None
        for off in range(P):
            if off + 1 < P:
                start_wcopy(off + 1)
            wcopies[off].wait()
            if off == 0 or _VARIANT == "local":
                a = xs_ref[pl.ds(me * MP, MP), :]
            else:
                rdmas[off - 1].wait_recv()
                a = buf_ref[off]
            wb = wbuf_ref[off % 2].astype(jnp.float8_e4m3fn)
            part = lax.dot_general(
                a, wb, (((1,), (0,)), ((), ())),
                preferred_element_type=jnp.float32)
            acc = part if acc is None else acc + part

        for rdma in rdmas:
            rdma.wait_send()

        y = acc * (sx_ref[0] * sw_ref[0])
        out_ref[...] = y * jax.nn.sigmoid(jnp.clip(y, -60.0, 60.0))

    return pl.pallas_call(
        body,
        out_shape=jax.ShapeDtypeStruct((MP, N), jnp.float32),
        in_specs=[
            pl.BlockSpec(memory_space=pltpu.VMEM),
            pl.BlockSpec(memory_space=pl.ANY),
            pl.BlockSpec(memory_space=pltpu.SMEM),
            pl.BlockSpec(memory_space=pltpu.SMEM),
        ],
        out_specs=pl.BlockSpec(memory_space=pltpu.VMEM),
        scratch_shapes=[
            pltpu.VMEM((M, KP), jnp.float8_e4m3fn),
            pltpu.VMEM((P, MP, KP), jnp.float8_e4m3fn),
            pltpu.VMEM((2, KP, N), jnp.float32),
            pltpu.SemaphoreType.DMA((P,)),
            pltpu.SemaphoreType.DMA((P,)),
            pltpu.SemaphoreType.DMA((2,)),
        ],
    )(x, w_mat, scale_x, scale_w)


# device time: 16195 ns/iter; 2.0330x vs baseline; 1.2074x over previous
import os

import jax
import jax.numpy as jnp
from jax import lax
from jax.experimental import pallas as pl
from jax.experimental.pallas import tpu as pltpu

_VARIANT = os.environ.get("KVARIANT", "full")

P = 16
M, K, N = 4096, 4096, 2048
MP = M // P
KP = K // P

ORDER = sorted(range(1, P), key=lambda o: (min(o, P - o), o))
NWBUF = 4


def kernel(x, w_mat, scale_x, scale_w):
    def body(x_ref, w_hbm, sx_ref, sw_ref, out_ref,
             xs_ref, buf_ref, wbuf_ref, send_sems, recv_sems, w_sems):
        me = lax.axis_index("i")
        steps = [0] + (ORDER if _VARIANT != "local" else list(range(1, P)))

        wcopies = []

        def start_wcopy(i):
            k0 = lax.rem(me - steps[i] + P, P)
            cp = pltpu.make_async_copy(
                w_hbm.at[pl.ds(k0 * KP, KP), :],
                wbuf_ref.at[i % NWBUF],
                w_sems.at[i % NWBUF],
            )
            cp.start()
            wcopies.append(cp)

        for i in range(NWBUF - 1):
            start_wcopy(i)
        xs_ref[...] = x_ref[...].astype(jnp.float8_e4m3fn)

        if _VARIANT != "local":
            barrier_sem = pltpu.get_barrier_semaphore()
            for off in range(1, P):
                pl.semaphore_signal(
                    barrier_sem, inc=1,
                    device_id=(lax.rem(me + off, P),),
                    device_id_type=pl.DeviceIdType.MESH,
                )
            pl.semaphore_wait(barrier_sem, P - 1)

        rdmas = {}
        for off in (ORDER if _VARIANT != "local" else []):
            tgt = lax.rem(me + off, P)
            rdma = pltpu.make_async_remote_copy(
                src_ref=xs_ref.at[pl.ds(tgt * MP, MP), :],
                dst_ref=buf_ref.at[off],
                send_sem=send_sems.at[off],
                recv_sem=recv_sems.at[off],
                device_id=(tgt,),
                device_id_type=pl.DeviceIdType.MESH,
            )
            rdma.start()
            rdmas[off] = rdma

        acc = None
        for i, off in enumerate(steps):
            if i + NWBUF - 1 < len(steps):
                start_wcopy(i + NWBUF - 1)
            wcopies[i].wait()
            if off == 0 or _VARIANT == "local":
                a = x_ref[pl.ds(me * MP, MP), :]
            else:
                rdmas[off].wait_recv()
                a = buf_ref[off].astype(jnp.float32)
            part = lax.dot_general(
                a, wbuf_ref[i % NWBUF],
                (((1,), (0,)), ((), ())),
                preferred_element_type=jnp.float32)
            acc = part if acc is None else acc + part

        for off in (ORDER if _VARIANT != "local" else []):
            rdmas[off].wait_send()

        y = acc * (sx_ref[0] * sw_ref[0])
        out_ref[...] = y * jax.nn.sigmoid(jnp.clip(y, -60.0, 60.0))

    return pl.pallas_call(
        body,
        out_shape=jax.ShapeDtypeStruct((MP, N), jnp.float32),
        in_specs=[
            pl.BlockSpec(memory_space=pltpu.VMEM),
            pl.BlockSpec(memory_space=pl.ANY),
            pl.BlockSpec(memory_space=pltpu.SMEM),
            pl.BlockSpec(memory_space=pltpu.SMEM),
        ],
        out_specs=pl.BlockSpec(memory_space=pltpu.VMEM),
        scratch_shapes=[
            pltpu.VMEM((M, KP), jnp.float8_e4m3fn),
            pltpu.VMEM((P, MP, KP), jnp.float8_e4m3fn),
            pltpu.VMEM((NWBUF, KP, N), jnp.float32),
            pltpu.SemaphoreType.DMA((P,)),
            pltpu.SemaphoreType.DMA((P,)),
            pltpu.SemaphoreType.DMA((NWBUF,)),
        ],
        compiler_params=pltpu.CompilerParams(
            collective_id=0 if _VARIANT != "local" else None),
    )(x, w_mat, scale_x, scale_w)
